# baseline (speedup 1.0000x reference)
import functools

import jax
import jax.numpy as jnp
from jax import lax
from jax.experimental import pallas as pl
from jax.experimental.pallas import tpu as pltpu
from jax.experimental.pallas import tpu_sc as plsc


def _gelu(x):
    return 0.5 * x * (1.0 + lax.erf(x * 0.7071067811865476))


_CHUNK = 128


def _make_sc_gather(R, seg):
    info = plsc.get_sparse_core_info()
    NW = info.num_cores * info.num_subcores
    NC = info.num_cores
    rows_per_w = R // NW
    nchunks = rows_per_w // _CHUNK
    npairs = nchunks // 2

    mesh = plsc.VectorSubcoreMesh(core_axis_name="c", subcore_axis_name="s")

    @functools.partial(
        pl.kernel,
        mesh=mesh,
        out_type=[jax.ShapeDtypeStruct((R, 128), jnp.float32),
                  jax.ShapeDtypeStruct((R, 128), jnp.float32)],
        scratch_types=(
            [pltpu.VMEM((rows_per_w,), jnp.int32) for _ in range(5)]
            + [pltpu.VMEM((_CHUNK, 128), jnp.float32) for _ in range(4)]
            + [pltpu.SemaphoreType.DMA, pltpu.SemaphoreType.DMA,
               pltpu.SemaphoreType.DMA, pltpu.SemaphoreType.DMA]
        ),
    )
    def sc_kernel(sp_idx2, mv_idx3,
                  spP, mv0P, mv1P, mv2P, mv3P,
                  hA, hB, sp_v, m0_v, m1_v, m2_v, m3_v,
                  hA0, hB0, hA1, hB1, bs0, bs1, as0, as1):
        wid = lax.axis_index("s") * NC + lax.axis_index("c")
        row_base = pl.multiple_of(wid * rows_per_w, rows_per_w)

        pltpu.sync_copy(sp_idx2.at[seg, pl.ds(row_base, rows_per_w)], sp_v)
        for k, idx_v in enumerate((m0_v, m1_v, m2_v, m3_v)):
            pltpu.sync_copy(mv_idx3.at[k, seg, pl.ds(row_base, rows_per_w)],
                            idx_v)

        bufs = ((hA0, hB0), (hA1, hB1))
        bsem = (bs0, bs1)
        asem = (as0, as1)

        def isl(idx_v, c):
            return idx_v.at[pl.ds(c * _CHUNK, _CHUNK)]

        def base_copies(c, s):
            hA_v, hB_v = bufs[s]
            return [pltpu.make_async_copy(spP.at[isl(sp_v, c)], hA_v,
                                          bsem[s]),
                    pltpu.make_async_copy(mv2P.at[isl(m2_v, c)], hB_v,
                                          bsem[s])]

        def acc_copies(c, s):
            hA_v, hB_v = bufs[s]
            return [pltpu.make_async_copy(mv0P.at[isl(m0_v, c)], hA_v,
                                          asem[s]),
                    pltpu.make_async_copy(mv1P.at[isl(m1_v, c)], hA_v,
                                          asem[s]),
                    pltpu.make_async_copy(mv3P.at[isl(m3_v, c)], hB_v,
                                          asem[s])]

        def finish(c, s):
            for cp in base_copies(c, s):
                cp.wait()
            accs = acc_copies(c, s)
            for cp in accs:
                cp.start(add=True)
            for cp in accs:
                cp.wait()
            hA_v, hB_v = bufs[s]
            row0 = row_base + c * _CHUNK
            pltpu.sync_copy(hA_v, hA.at[pl.ds(row0, _CHUNK)])
            pltpu.sync_copy(hB_v, hB.at[pl.ds(row0, _CHUNK)])

        for cp in base_copies(0, 0):
            cp.start()

        def body(p, carry):
            c0 = p * 2
            for cp in base_copies(c0 + 1, 1):
                cp.start()
            finish(c0, 0)

            @pl.when(c0 + 2 < nchunks)
            def _():
                for cp in base_copies(c0 + 2, 0):
                    cp.start()

            finish(c0 + 1, 1)
            return carry

        lax.fori_loop(0, npairs, body, 0)

    return sc_kernel


def _mlp_body(hA_ref, hB_ref, ids_ref, scal_ref,
              itT_ref, abT_ref, stT_ref, w1ab_ref, w1t_ref, b1_ref,
              w2_ref, b2_ref, o_ref):
    blk = hA_ref.shape[0]

    def onehot_emb(ids, tab_ref):
        v = tab_ref.shape[0]
        oh = (ids.astype(jnp.int16)[:, None]
              == lax.broadcasted_iota(jnp.int16, (blk, v), 1))
        return jnp.dot(oh.astype(jnp.bfloat16), tab_ref[...],
                       preferred_element_type=jnp.float32)

    tail = jnp.concatenate(
        [onehot_emb(ids_ref[0, :], itT_ref),
         onehot_emb(ids_ref[1, :], abT_ref),
         onehot_emb(ids_ref[2, :], stT_ref),
         scal_ref[...].T], axis=1)
    xab = jnp.concatenate([hA_ref[...], hB_ref[...]], axis=1)
    h = (jnp.dot(xab.astype(jnp.bfloat16), w1ab_ref[...],
                 preferred_element_type=jnp.float32)
         + jnp.dot(tail.astype(jnp.bfloat16), w1t_ref[...],
                   preferred_element_type=jnp.float32))
    h = _gelu(h + b1_ref[...])
    o = jnp.dot(h.astype(jnp.bfloat16), w2_ref[...],
                preferred_element_type=jnp.float32)
    o_ref[0, :, :] = _gelu(o + b2_ref[...])


def _mlp_body_chain(hA_ref, hB_ref, ids_ref, scal_ref, itT_ref, abT_ref,
                    stT_ref, w1ab_ref, w1t_ref, b1_ref, w2_ref, b2_ref,
                    prev_ref, o_ref):
    del prev_ref
    _mlp_body(hA_ref, hB_ref, ids_ref, scal_ref, itT_ref, abT_ref, stT_ref,
              w1ab_ref, w1t_ref, b1_ref, w2_ref, b2_ref, o_ref)


def kernel(species_ids, move_ids, item_ids, ability_ids, status_ids,
           hp_values, boost_values, mega_flags, species_table, move_table,
           item_table, ability_table, status_table, W1, b1, W2, b2):
    B, N = species_ids.shape
    R = B * N
    IN_D, HIDDEN = W1.shape
    OUT_D = W2.shape[1]
    SP_D = species_table.shape[1]
    MV_D = move_table.shape[1]
    IT_D = item_table.shape[1]
    AB_D = ability_table.shape[1]
    ST_D = status_table.shape[1]

    def flat(ids):
        return ids.T.reshape(R).astype(jnp.int32)

    sp2 = species_ids.astype(jnp.int32).T
    mv3 = move_ids.astype(jnp.int32).transpose(2, 1, 0)

    def pad_band(tab, lo, width=128):
        v, d = tab.shape
        return jnp.concatenate(
            [jnp.zeros((v, lo), jnp.float32), tab,
             jnp.zeros((v, width - lo - d), jnp.float32)], axis=1)

    spP = pad_band(species_table, 0)
    mv0P = pad_band(move_table, SP_D)
    mv1P = pad_band(move_table, SP_D + MV_D)
    mv2P = pad_band(move_table, 0)
    mv3P = pad_band(move_table, MV_D)

    W1ab = jnp.concatenate(
        [W1[:SP_D + 2 * MV_D], W1[SP_D + 2 * MV_D:SP_D + 4 * MV_D],
         jnp.zeros((64, HIDDEN), jnp.float32)], axis=0)
    W1tail = W1[SP_D + 4 * MV_D:]

    SEG = N
    Rseg = R // SEG

    ids2 = jnp.stack([flat(item_ids), flat(ability_ids), flat(status_ids)])
    scal8 = jnp.concatenate(
        [hp_values.T.reshape(1, R),
         boost_values.transpose(2, 1, 0).reshape(6, R),
         mega_flags.T.reshape(1, R)], axis=0)

    itT = item_table.astype(jnp.bfloat16)
    abT = ability_table.astype(jnp.bfloat16)
    stT = status_table.astype(jnp.bfloat16)
    W1ab16 = W1ab.astype(jnp.bfloat16)
    W1t16 = W1tail.astype(jnp.bfloat16)
    W216 = W2.astype(jnp.bfloat16)
    b1r = b1.reshape(1, HIDDEN)
    b2r = b2.reshape(1, OUT_D)

    BLK = min(1024, Rseg)
    GRD = Rseg // BLK

    out = None
    for s in range(SEG):
        hA, hB = _make_sc_gather(Rseg, s)(
            sp2, mv3, spP, mv0P, mv1P, mv2P, mv3P)

        off = s * (Rseg // BLK)
        in_specs = [
            pl.BlockSpec((BLK, 128), lambda i: (i, 0)),
            pl.BlockSpec((BLK, 128), lambda i: (i, 0)),
            pl.BlockSpec((3, BLK), lambda i, o=off: (0, o + i)),
            pl.BlockSpec((8, BLK), lambda i, o=off: (0, o + i)),
            pl.BlockSpec(item_table.shape, lambda i: (0, 0)),
            pl.BlockSpec(ability_table.shape, lambda i: (0, 0)),
            pl.BlockSpec(status_table.shape, lambda i: (0, 0)),
            pl.BlockSpec((IN_D, HIDDEN), lambda i: (0, 0)),
            pl.BlockSpec((IT_D + AB_D + ST_D + 8, HIDDEN),
                         lambda i: (0, 0)),
            pl.BlockSpec((1, HIDDEN), lambda i: (0, 0)),
            pl.BlockSpec((HIDDEN, OUT_D), lambda i: (0, 0)),
            pl.BlockSpec((1, OUT_D), lambda i: (0, 0)),
        ]
        operands = [hA, hB, ids2, scal8,
                    itT, abT, stT, W1ab16, W1t16, b1r, W216, b2r]
        kwargs = {}
        body = _mlp_body
        if s > 0:
            body = _mlp_body_chain
            in_specs.append(pl.BlockSpec(memory_space=pl.ANY))
            operands.append(out)
            kwargs["input_output_aliases"] = {12: 0}
        out = pl.pallas_call(
            body,
            grid=(GRD,),
            in_specs=in_specs,
            out_specs=pl.BlockSpec((1, BLK, OUT_D), lambda i, s=s: (s, i, 0)),
            out_shape=jax.ShapeDtypeStruct((N, B, OUT_D), jnp.float32),
            **kwargs,
        )(*operands)

    return out.transpose(1, 0, 2)


# speedup vs baseline: 1.0117x; 1.0117x over previous
import functools

import jax
import jax.numpy as jnp
from jax import lax
from jax.experimental import pallas as pl
from jax.experimental.pallas import tpu as pltpu
from jax.experimental.pallas import tpu_sc as plsc


def _gelu(x):
    return 0.5 * x * (1.0 + lax.erf(x * 0.7071067811865476))


_CHUNK = 128


def _make_sc_gather(R, seg):
    info = plsc.get_sparse_core_info()
    NW = info.num_cores * info.num_subcores
    NC = info.num_cores
    rows_per_w = R // NW
    nchunks = rows_per_w // _CHUNK
    npairs = nchunks // 2

    mesh = plsc.VectorSubcoreMesh(core_axis_name="c", subcore_axis_name="s")

    @functools.partial(
        pl.kernel,
        mesh=mesh,
        out_type=[jax.ShapeDtypeStruct((R, 128), jnp.float32),
                  jax.ShapeDtypeStruct((R, 128), jnp.float32)],
        scratch_types=(
            [pltpu.VMEM((rows_per_w,), jnp.int32) for _ in range(5)]
            + [pltpu.VMEM((_CHUNK, 128), jnp.float32) for _ in range(4)]
            + [pltpu.SemaphoreType.DMA, pltpu.SemaphoreType.DMA,
               pltpu.SemaphoreType.DMA, pltpu.SemaphoreType.DMA]
        ),
    )
    def sc_kernel(sp_idx2, mv_idx3,
                  spP, mv0P, mv1P, mv2P, mv3P,
                  hA, hB, sp_v, m0_v, m1_v, m2_v, m3_v,
                  hA0, hB0, hA1, hB1, bs0, bs1, as0, as1):
        wid = lax.axis_index("s") * NC + lax.axis_index("c")
        row_base = pl.multiple_of(wid * rows_per_w, rows_per_w)

        pltpu.sync_copy(sp_idx2.at[seg, pl.ds(row_base, rows_per_w)], sp_v)
        for k, idx_v in enumerate((m0_v, m1_v, m2_v, m3_v)):
            pltpu.sync_copy(mv_idx3.at[k, seg, pl.ds(row_base, rows_per_w)],
                            idx_v)

        bufs = ((hA0, hB0), (hA1, hB1))
        bsem = (bs0, bs1)
        asem = (as0, as1)

        def isl(idx_v, c):
            return idx_v.at[pl.ds(c * _CHUNK, _CHUNK)]

        def base_copies(c, s):
            hA_v, hB_v = bufs[s]
            return [pltpu.make_async_copy(spP.at[isl(sp_v, c)], hA_v,
                                          bsem[s]),
                    pltpu.make_async_copy(mv2P.at[isl(m2_v, c)], hB_v,
                                          bsem[s])]

        def acc_copies(c, s):
            hA_v, hB_v = bufs[s]
            return [pltpu.make_async_copy(mv0P.at[isl(m0_v, c)], hA_v,
                                          asem[s]),
                    pltpu.make_async_copy(mv1P.at[isl(m1_v, c)], hA_v,
                                          asem[s]),
                    pltpu.make_async_copy(mv3P.at[isl(m3_v, c)], hB_v,
                                          asem[s])]

        def finish(c, s):
            for cp in base_copies(c, s):
                cp.wait()
            accs = acc_copies(c, s)
            for cp in accs:
                cp.start(add=True)
            for cp in accs:
                cp.wait()
            hA_v, hB_v = bufs[s]
            row0 = row_base + c * _CHUNK
            pltpu.sync_copy(hA_v, hA.at[pl.ds(row0, _CHUNK)])
            pltpu.sync_copy(hB_v, hB.at[pl.ds(row0, _CHUNK)])

        for cp in base_copies(0, 0):
            cp.start()

        def body(p, carry):
            c0 = p * 2
            for cp in base_copies(c0 + 1, 1):
                cp.start()
            finish(c0, 0)

            @pl.when(c0 + 2 < nchunks)
            def _():
                for cp in base_copies(c0 + 2, 0):
                    cp.start()

            finish(c0 + 1, 1)
            return carry

        lax.fori_loop(0, npairs, body, 0)

    return sc_kernel


def _mlp_body(hA_ref, hB_ref, ids_ref, scal_ref,
              itT_ref, abT_ref, stT_ref, w1ab_ref, w1t_ref, b1_ref,
              w2_ref, b2_ref, o_ref):
    blk = hA_ref.shape[0]

    def onehot_emb(ids, tab_ref):
        v = tab_ref.shape[0]
        oh = (ids[:, None] == lax.broadcasted_iota(jnp.int32, (blk, v), 1))
        return jnp.dot(oh.astype(jnp.bfloat16), tab_ref[...],
                       preferred_element_type=jnp.float32)

    tail = jnp.concatenate(
        [onehot_emb(ids_ref[0, :], itT_ref),
         onehot_emb(ids_ref[1, :], abT_ref),
         onehot_emb(ids_ref[2, :], stT_ref),
         scal_ref[...].T], axis=1)
    xab = jnp.concatenate([hA_ref[...], hB_ref[...]], axis=1)
    h = (jnp.dot(xab.astype(jnp.bfloat16), w1ab_ref[...],
                 preferred_element_type=jnp.float32)
         + jnp.dot(tail.astype(jnp.bfloat16), w1t_ref[...],
                   preferred_element_type=jnp.float32))
    h = _gelu(h + b1_ref[...])
    o = jnp.dot(h.astype(jnp.bfloat16), w2_ref[...],
                preferred_element_type=jnp.float32)
    o_ref[0, :, :] = _gelu(o + b2_ref[...])


def _mlp_body_chain(hA_ref, hB_ref, ids_ref, scal_ref, itT_ref, abT_ref,
                    stT_ref, w1ab_ref, w1t_ref, b1_ref, w2_ref, b2_ref,
                    prev_ref, o_ref):
    del prev_ref
    _mlp_body(hA_ref, hB_ref, ids_ref, scal_ref, itT_ref, abT_ref, stT_ref,
              w1ab_ref, w1t_ref, b1_ref, w2_ref, b2_ref, o_ref)


def kernel(species_ids, move_ids, item_ids, ability_ids, status_ids,
           hp_values, boost_values, mega_flags, species_table, move_table,
           item_table, ability_table, status_table, W1, b1, W2, b2):
    B, N = species_ids.shape
    R = B * N
    IN_D, HIDDEN = W1.shape
    OUT_D = W2.shape[1]
    SP_D = species_table.shape[1]
    MV_D = move_table.shape[1]
    IT_D = item_table.shape[1]
    AB_D = ability_table.shape[1]
    ST_D = status_table.shape[1]

    def flat(ids):
        return ids.T.reshape(R).astype(jnp.int32)

    sp2 = species_ids.astype(jnp.int32).T
    mv3 = move_ids.astype(jnp.int32).transpose(2, 1, 0)

    def pad_band(tab, lo, width=128):
        v, d = tab.shape
        return jnp.concatenate(
            [jnp.zeros((v, lo), jnp.float32), tab,
             jnp.zeros((v, width - lo - d), jnp.float32)], axis=1)

    spP = pad_band(species_table, 0)
    mv0P = pad_band(move_table, SP_D)
    mv1P = pad_band(move_table, SP_D + MV_D)
    mv2P = pad_band(move_table, 0)
    mv3P = pad_band(move_table, MV_D)

    W1ab = jnp.concatenate(
        [W1[:SP_D + 2 * MV_D], W1[SP_D + 2 * MV_D:SP_D + 4 * MV_D],
         jnp.zeros((64, HIDDEN), jnp.float32)], axis=0)
    W1tail = W1[SP_D + 4 * MV_D:]

    SEG = N
    Rseg = R // SEG

    ids2 = jnp.stack([flat(item_ids), flat(ability_ids), flat(status_ids)])
    scal8 = jnp.concatenate(
        [hp_values.T.reshape(1, R),
         boost_values.transpose(2, 1, 0).reshape(6, R),
         mega_flags.T.reshape(1, R)], axis=0)

    itT = item_table.astype(jnp.bfloat16)
    abT = ability_table.astype(jnp.bfloat16)
    stT = status_table.astype(jnp.bfloat16)
    W1ab16 = W1ab.astype(jnp.bfloat16)
    W1t16 = W1tail.astype(jnp.bfloat16)
    W216 = W2.astype(jnp.bfloat16)
    b1r = b1.reshape(1, HIDDEN)
    b2r = b2.reshape(1, OUT_D)

    BLK = min(1024, Rseg)
    GRD = Rseg // BLK

    out = None
    for s in range(SEG):
        hA, hB = _make_sc_gather(Rseg, s)(
            sp2, mv3, spP, mv0P, mv1P, mv2P, mv3P)

        off = s * (Rseg // BLK)
        in_specs = [
            pl.BlockSpec((BLK, 128), lambda i: (i, 0)),
            pl.BlockSpec((BLK, 128), lambda i: (i, 0)),
            pl.BlockSpec((3, BLK), lambda i, o=off: (0, o + i)),
            pl.BlockSpec((8, BLK), lambda i, o=off: (0, o + i)),
            pl.BlockSpec(item_table.shape, lambda i: (0, 0)),
            pl.BlockSpec(ability_table.shape, lambda i: (0, 0)),
            pl.BlockSpec(status_table.shape, lambda i: (0, 0)),
            pl.BlockSpec((IN_D, HIDDEN), lambda i: (0, 0)),
            pl.BlockSpec((IT_D + AB_D + ST_D + 8, HIDDEN),
                         lambda i: (0, 0)),
            pl.BlockSpec((1, HIDDEN), lambda i: (0, 0)),
            pl.BlockSpec((HIDDEN, OUT_D), lambda i: (0, 0)),
            pl.BlockSpec((1, OUT_D), lambda i: (0, 0)),
        ]
        operands = [hA, hB, ids2, scal8,
                    itT, abT, stT, W1ab16, W1t16, b1r, W216, b2r]
        kwargs = {}
        body = _mlp_body
        if s > 0:
            body = _mlp_body_chain
            in_specs.append(pl.BlockSpec(memory_space=pl.ANY))
            operands.append(out)
            kwargs["input_output_aliases"] = {12: 0}
        out = pl.pallas_call(
            body,
            grid=(GRD,),
            in_specs=in_specs,
            out_specs=pl.BlockSpec((1, BLK, OUT_D), lambda i, s=s: (s, i, 0)),
            out_shape=jax.ShapeDtypeStruct((N, B, OUT_D), jnp.float32),
            **kwargs,
        )(*operands)

    return out.transpose(1, 0, 2)


# speedup vs baseline: 1.0196x; 1.0078x over previous
import functools

import jax
import jax.numpy as jnp
from jax import lax
from jax.experimental import pallas as pl
from jax.experimental.pallas import tpu as pltpu
from jax.experimental.pallas import tpu_sc as plsc


def _gelu(x):
    return 0.5 * x * (1.0 + lax.erf(x * 0.7071067811865476))


_CHUNK = 128


def _make_sc_gather(R, seg):
    info = plsc.get_sparse_core_info()
    NW = info.num_cores * info.num_subcores
    NC = info.num_cores
    rows_per_w = R // NW
    nchunks = rows_per_w // _CHUNK
    npairs = nchunks // 2

    mesh = plsc.VectorSubcoreMesh(core_axis_name="c", subcore_axis_name="s")

    @functools.partial(
        pl.kernel,
        mesh=mesh,
        out_type=[jax.ShapeDtypeStruct((R, 128), jnp.float32),
                  jax.ShapeDtypeStruct((R, 128), jnp.float32)],
        scratch_types=(
            [pltpu.VMEM((rows_per_w,), jnp.int32) for _ in range(5)]
            + [pltpu.VMEM((_CHUNK, 128), jnp.float32) for _ in range(4)]
            + [pltpu.SemaphoreType.DMA, pltpu.SemaphoreType.DMA,
               pltpu.SemaphoreType.DMA, pltpu.SemaphoreType.DMA]
        ),
    )
    def sc_kernel(sp_idx2, mv_idx3,
                  spP, mv0P, mv1P, mv2P, mv3P,
                  hA, hB, sp_v, m0_v, m1_v, m2_v, m3_v,
                  hA0, hB0, hA1, hB1, bs0, bs1, as0, as1):
        wid = lax.axis_index("s") * NC + lax.axis_index("c")
        row_base = pl.multiple_of(wid * rows_per_w, rows_per_w)

        pltpu.sync_copy(sp_idx2.at[seg, pl.ds(row_base, rows_per_w)], sp_v)
        for k, idx_v in enumerate((m0_v, m1_v, m2_v, m3_v)):
            pltpu.sync_copy(mv_idx3.at[k, seg, pl.ds(row_base, rows_per_w)],
                            idx_v)

        bufs = ((hA0, hB0), (hA1, hB1))
        bsem = (bs0, bs1)
        asem = (as0, as1)

        def isl(idx_v, c):
            return idx_v.at[pl.ds(c * _CHUNK, _CHUNK)]

        def base_copies(c, s):
            hA_v, hB_v = bufs[s]
            return [pltpu.make_async_copy(spP.at[isl(sp_v, c)], hA_v,
                                          bsem[s]),
                    pltpu.make_async_copy(mv2P.at[isl(m2_v, c)], hB_v,
                                          bsem[s])]

        def acc_copies(c, s):
            hA_v, hB_v = bufs[s]
            return [pltpu.make_async_copy(mv0P.at[isl(m0_v, c)], hA_v,
                                          asem[s]),
                    pltpu.make_async_copy(mv1P.at[isl(m1_v, c)], hA_v,
                                          asem[s]),
                    pltpu.make_async_copy(mv3P.at[isl(m3_v, c)], hB_v,
                                          asem[s])]

        def finish(c, s):
            for cp in base_copies(c, s):
                cp.wait()
            accs = acc_copies(c, s)
            for cp in accs:
                cp.start(add=True)
            for cp in accs:
                cp.wait()
            hA_v, hB_v = bufs[s]
            row0 = row_base + c * _CHUNK
            pltpu.sync_copy(hA_v, hA.at[pl.ds(row0, _CHUNK)])
            pltpu.sync_copy(hB_v, hB.at[pl.ds(row0, _CHUNK)])

        for cp in base_copies(0, 0):
            cp.start()

        def body(p, carry):
            c0 = p * 2
            for cp in base_copies(c0 + 1, 1):
                cp.start()
            finish(c0, 0)

            @pl.when(c0 + 2 < nchunks)
            def _():
                for cp in base_copies(c0 + 2, 0):
                    cp.start()

            finish(c0 + 1, 1)
            return carry

        lax.fori_loop(0, npairs, body, 0)

    return sc_kernel


def _mlp_body(hA_ref, hB_ref, ids_ref, scal_ref,
              itT_ref, abT_ref, stT_ref, w1ab_ref, w1t_ref, b1_ref,
              w2_ref, b2_ref, o_ref):
    blk = hA_ref.shape[0]

    def onehot_emb(ids, tab_ref):
        v = tab_ref.shape[0]
        oh = (ids[:, None] == lax.broadcasted_iota(jnp.int32, (blk, v), 1))
        return jnp.dot(oh.astype(jnp.bfloat16), tab_ref[...],
                       preferred_element_type=jnp.float32)

    tail = jnp.concatenate(
        [onehot_emb(ids_ref[0, :], itT_ref),
         onehot_emb(ids_ref[1, :], abT_ref),
         onehot_emb(ids_ref[2, :], stT_ref),
         scal_ref[...].T], axis=1)
    xab = jnp.concatenate([hA_ref[...], hB_ref[...]], axis=1)
    h = (jnp.dot(xab.astype(jnp.bfloat16), w1ab_ref[...],
                 preferred_element_type=jnp.float32)
         + jnp.dot(tail.astype(jnp.bfloat16), w1t_ref[...],
                   preferred_element_type=jnp.float32))
    h = _gelu(h + b1_ref[...])
    o = jnp.dot(h.astype(jnp.bfloat16), w2_ref[...],
                preferred_element_type=jnp.float32)
    o_ref[0, :, :] = _gelu(o + b2_ref[...])


def _mlp_body_chain(hA_ref, hB_ref, ids_ref, scal_ref, itT_ref, abT_ref,
                    stT_ref, w1ab_ref, w1t_ref, b1_ref, w2_ref, b2_ref,
                    prev_ref, o_ref):
    del prev_ref
    _mlp_body(hA_ref, hB_ref, ids_ref, scal_ref, itT_ref, abT_ref, stT_ref,
              w1ab_ref, w1t_ref, b1_ref, w2_ref, b2_ref, o_ref)


def kernel(species_ids, move_ids, item_ids, ability_ids, status_ids,
           hp_values, boost_values, mega_flags, species_table, move_table,
           item_table, ability_table, status_table, W1, b1, W2, b2):
    B, N = species_ids.shape
    R = B * N
    IN_D, HIDDEN = W1.shape
    OUT_D = W2.shape[1]
    SP_D = species_table.shape[1]
    MV_D = move_table.shape[1]
    IT_D = item_table.shape[1]
    AB_D = ability_table.shape[1]
    ST_D = status_table.shape[1]

    def flat(ids):
        return ids.T.reshape(R).astype(jnp.int32)

    sp2 = species_ids.astype(jnp.int32).T
    mv3 = move_ids.astype(jnp.int32).transpose(2, 1, 0)

    def pad_band(tab, lo, width=128):
        v, d = tab.shape
        return jnp.concatenate(
            [jnp.zeros((v, lo), jnp.float32), tab,
             jnp.zeros((v, width - lo - d), jnp.float32)], axis=1)

    spP = pad_band(species_table, 0)
    mv0P = pad_band(move_table, SP_D)
    mv1P = pad_band(move_table, SP_D + MV_D)
    mv2P = pad_band(move_table, 0)
    mv3P = pad_band(move_table, MV_D)

    W1ab = jnp.concatenate(
        [W1[:SP_D + 2 * MV_D], W1[SP_D + 2 * MV_D:SP_D + 4 * MV_D],
         jnp.zeros((64, HIDDEN), jnp.float32)], axis=0)
    W1tail = W1[SP_D + 4 * MV_D:]

    SEG = N
    Rseg = R // SEG

    ids2 = jnp.stack([flat(item_ids), flat(ability_ids), flat(status_ids)])
    scal8 = jnp.concatenate(
        [hp_values.T.reshape(1, R),
         boost_values.transpose(2, 1, 0).reshape(6, R),
         mega_flags.T.reshape(1, R)], axis=0)

    itT = item_table.astype(jnp.bfloat16)
    abT = ability_table.astype(jnp.bfloat16)
    stT = status_table.astype(jnp.bfloat16)
    W1ab16 = W1ab.astype(jnp.bfloat16)
    W1t16 = W1tail.astype(jnp.bfloat16)
    W216 = W2.astype(jnp.bfloat16)
    b1r = b1.reshape(1, HIDDEN)
    b2r = b2.reshape(1, OUT_D)

    BLK = min(2048, Rseg)
    GRD = Rseg // BLK

    out = None
    for s in range(SEG):
        hA, hB = _make_sc_gather(Rseg, s)(
            sp2, mv3, spP, mv0P, mv1P, mv2P, mv3P)

        off = s * (Rseg // BLK)
        in_specs = [
            pl.BlockSpec((BLK, 128), lambda i: (i, 0)),
            pl.BlockSpec((BLK, 128), lambda i: (i, 0)),
            pl.BlockSpec((3, BLK), lambda i, o=off: (0, o + i)),
            pl.BlockSpec((8, BLK), lambda i, o=off: (0, o + i)),
            pl.BlockSpec(item_table.shape, lambda i: (0, 0)),
            pl.BlockSpec(ability_table.shape, lambda i: (0, 0)),
            pl.BlockSpec(status_table.shape, lambda i: (0, 0)),
            pl.BlockSpec((IN_D, HIDDEN), lambda i: (0, 0)),
            pl.BlockSpec((IT_D + AB_D + ST_D + 8, HIDDEN),
                         lambda i: (0, 0)),
            pl.BlockSpec((1, HIDDEN), lambda i: (0, 0)),
            pl.BlockSpec((HIDDEN, OUT_D), lambda i: (0, 0)),
            pl.BlockSpec((1, OUT_D), lambda i: (0, 0)),
        ]
        operands = [hA, hB, ids2, scal8,
                    itT, abT, stT, W1ab16, W1t16, b1r, W216, b2r]
        kwargs = {}
        body = _mlp_body
        if s > 0:
            body = _mlp_body_chain
            in_specs.append(pl.BlockSpec(memory_space=pl.ANY))
            operands.append(out)
            kwargs["input_output_aliases"] = {12: 0}
        out = pl.pallas_call(
            body,
            grid=(GRD,),
            in_specs=in_specs,
            out_specs=pl.BlockSpec((1, BLK, OUT_D), lambda i, s=s: (s, i, 0)),
            out_shape=jax.ShapeDtypeStruct((N, B, OUT_D), jnp.float32),
            **kwargs,
        )(*operands)

    return out.transpose(1, 0, 2)
